# baseline (speedup 1.0000x reference)
import jax
import jax.numpy as jnp
from jax.experimental import pallas as pl
from jax.experimental.pallas import tpu as pltpu

_K1 = 31
_LO0 = -2139095042
_HI0 = 2139095041


def _key_to_f32(m):
    return jax.lax.bitcast_convert_type(
        jnp.where(m < 0, m ^ jnp.int32(0x7FFFFFFF), m), jnp.float32
    )


_NSTREAM = 2


def _kth_stat(a, idx, axis, ji_ref):
    n = a.shape[axis]
    b = 1 - axis
    half = a.shape[b] // _NSTREAM

    def bslice(x, p):
        sl = slice(p * half, (p + 1) * half)
        return x[sl, :] if b == 0 else x[:, sl]

    parts = [bslice(a, p) for p in range(_NSTREAM)]
    shapes = [
        tuple(1 if d == axis else s for d, s in enumerate(pa.shape)) for pa in parts
    ]

    init = []
    for sh in shapes:
        init += [
            jnp.full(sh, _LO0, jnp.int32),
            jnp.full(sh, _HI0, jnp.int32),
            jnp.full(sh, n, jnp.int32),
            jnp.zeros(sh, jnp.int32),
        ]

    def vbody(_, carry):
        out = []
        for p, pa in enumerate(parts):
            lo, hi, clo, chi = carry[4 * p : 4 * p + 4]
            mid = (lo >> 1) + (hi >> 1) + (lo & hi & 1)
            c = jnp.sum(
                pa >= _key_to_f32(mid), axis=axis, keepdims=True, dtype=jnp.int32
            )
            big = c >= _K1
            out += [
                jnp.where(big, mid, lo),
                jnp.where(big, hi, mid),
                jnp.where(big, c, clo),
                jnp.where(big, chi, c),
            ]
        return tuple(out)

    res = jax.lax.fori_loop(0, 32, vbody, tuple(init))

    ts = []
    for p, (pa, sh) in enumerate(zip(parts, shapes)):
        lo, hi, clo, chi = res[4 * p : 4 * p + 4]
        t = _key_to_f32(lo)
        s = _K1 - chi
        pidx = bslice(idx, p)
        sl = slice(p * half, (p + 1) * half)
        jsl = (sl, slice(None)) if b == 0 else (slice(None), sl)

        ji_ref[jsl] = jnp.full(sh, n - 1, jnp.int32)
        split = clo > _K1

        @pl.when(jnp.any(split))
        def _slow_ji(pa=pa, pidx=pidx, sh=sh, t=t, s=s, split=split, jsl=jsl):
            masked_idx = jnp.where(pa == t, pidx, n)
            jlo = jnp.full(sh, -1, jnp.int32)
            jhi = jnp.full(sh, n - 1, jnp.int32)

            def ibody(_, carry):
                jlo, jhi = carry
                mid = jlo + ((jhi - jlo) >> 1)
                c = jnp.sum(
                    masked_idx <= mid, axis=axis, keepdims=True, dtype=jnp.int32
                )
                pred = c >= s
                return jnp.where(pred, jlo, mid), jnp.where(pred, mid, jhi)

            nbits = max(1, (n - 1).bit_length())
            jlo, jhi = jax.lax.fori_loop(0, nbits, ibody, (jlo, jhi))
            ji_ref[jsl] = jnp.where(split, jhi, n - 1)

        ts.append(t)

    return jnp.concatenate(ts, axis=b)


def _row_thr_kernel(a_ref, t_ref, ji_ref):
    a = a_ref[...]
    idx = jax.lax.broadcasted_iota(jnp.int32, a.shape, 1)
    t_ref[...] = _kth_stat(a, idx, 1, ji_ref)


def _col_mask_kernel(a_ref, rt_ref, rji_ref, out_ref, cji_ref):
    a = a_ref[...]
    c = a.shape[1]
    ridx = jax.lax.broadcasted_iota(jnp.int32, a.shape, 0)
    ct = _kth_stat(a, ridx, 0, cji_ref)
    cji = cji_ref[...]

    rt = rt_ref[...]
    rji = rji_ref[...]
    cidx = jax.lax.broadcasted_iota(jnp.int32, a.shape, 1) + pl.program_id(0) * c
    keep_r = (a > rt) | ((a == rt) & (cidx <= rji))
    keep_c = (a > ct) | ((a == ct) & (ridx <= cji))
    keep = keep_r & keep_c & (ridx != cidx)
    out_ref[...] = jnp.where(keep, a, 0.0)


def kernel(affinity):
    n = affinity.shape[0]
    blk_r = 512
    blk_c = 512

    rt, rji = pl.pallas_call(
        _row_thr_kernel,
        grid=(n // blk_r,),
        in_specs=[pl.BlockSpec((blk_r, n), lambda i: (i, 0))],
        out_specs=[
            pl.BlockSpec((blk_r, 1), lambda i: (i, 0)),
            pl.BlockSpec((blk_r, 1), lambda i: (i, 0)),
        ],
        out_shape=[
            jax.ShapeDtypeStruct((n, 1), affinity.dtype),
            jax.ShapeDtypeStruct((n, 1), jnp.int32),
        ],
    )(affinity)

    out, _ = pl.pallas_call(
        _col_mask_kernel,
        grid=(n // blk_c,),
        in_specs=[
            pl.BlockSpec((n, blk_c), lambda j: (0, j)),
            pl.BlockSpec((n, 1), lambda j: (0, 0)),
            pl.BlockSpec((n, 1), lambda j: (0, 0)),
        ],
        out_specs=[
            pl.BlockSpec((n, blk_c), lambda j: (0, j)),
            pl.BlockSpec((1, blk_c), lambda j: (0, j)),
        ],
        out_shape=[
            jax.ShapeDtypeStruct((n, n), affinity.dtype),
            jax.ShapeDtypeStruct((1, n), jnp.int32),
        ],
    )(affinity, rt, rji)

    return out


# speedup vs baseline: 1.0246x; 1.0246x over previous
import jax
import jax.numpy as jnp
from jax.experimental import pallas as pl
from jax.experimental.pallas import tpu as pltpu

_K1 = 31
_LO0 = -2139095042
_HI0 = 2139095041


def _key_to_f32(m):
    return jax.lax.bitcast_convert_type(
        jnp.where(m < 0, m ^ jnp.int32(0x7FFFFFFF), m), jnp.float32
    )


def _kth_stat(a, idx, axis, ji_ref):
    n = a.shape[axis]
    shape1 = tuple(1 if d == axis else s for d, s in enumerate(a.shape))
    lo = jnp.full(shape1, _LO0, jnp.int32)
    hi = jnp.full(shape1, _HI0, jnp.int32)
    clo = jnp.full(shape1, n, jnp.int32)
    chi = jnp.zeros(shape1, jnp.int32)

    def vbody(_, carry):
        lo, hi, clo, chi = carry
        mid = (lo >> 1) + (hi >> 1) + (lo & hi & 1)
        c = jnp.sum(a >= _key_to_f32(mid), axis=axis, keepdims=True, dtype=jnp.int32)
        big = c >= _K1
        return (
            jnp.where(big, mid, lo),
            jnp.where(big, hi, mid),
            jnp.where(big, c, clo),
            jnp.where(big, chi, c),
        )

    lo, hi, clo, chi = jax.lax.fori_loop(0, 32, vbody, (lo, hi, clo, chi))
    t = _key_to_f32(lo)
    s = _K1 - chi

    ji_ref[...] = jnp.full(shape1, n - 1, jnp.int32)
    split = clo > _K1

    @pl.when(jnp.any(split))
    def _slow_ji():
        masked_idx = jnp.where(a == t, idx, n)
        jlo = jnp.full(shape1, -1, jnp.int32)
        jhi = jnp.full(shape1, n - 1, jnp.int32)

        def ibody(_, carry):
            jlo, jhi = carry
            mid = jlo + ((jhi - jlo) >> 1)
            c = jnp.sum(masked_idx <= mid, axis=axis, keepdims=True, dtype=jnp.int32)
            pred = c >= s
            return jnp.where(pred, jlo, mid), jnp.where(pred, mid, jhi)

        nbits = max(1, (n - 1).bit_length())
        jlo, jhi = jax.lax.fori_loop(0, nbits, ibody, (jlo, jhi))
        ji_ref[...] = jnp.where(split, jhi, n - 1)

    return t


def _row_thr_kernel(a_ref, t_ref, ji_ref):
    a = a_ref[...]
    idx = jax.lax.broadcasted_iota(jnp.int32, a.shape, 1)
    t_ref[...] = _kth_stat(a, idx, 1, ji_ref)


def _col_mask_kernel(a_ref, rt_ref, rji_ref, out_ref, cji_ref):
    a = a_ref[...]
    c = a.shape[1]
    ridx = jax.lax.broadcasted_iota(jnp.int32, a.shape, 0)
    ct = _kth_stat(a, ridx, 0, cji_ref)
    cji = cji_ref[...]

    rt = rt_ref[...]
    rji = rji_ref[...]
    cidx = jax.lax.broadcasted_iota(jnp.int32, a.shape, 1) + pl.program_id(0) * c
    keep_r = (a > rt) | ((a == rt) & (cidx <= rji))
    keep_c = (a > ct) | ((a == ct) & (ridx <= cji))
    keep = keep_r & keep_c & (ridx != cidx)
    out_ref[...] = jnp.where(keep, a, 0.0)


def kernel(affinity):
    n = affinity.shape[0]
    blk_r = 512
    blk_c = 512

    rt, rji = pl.pallas_call(
        _row_thr_kernel,
        grid=(n // blk_r,),
        in_specs=[pl.BlockSpec((blk_r, n), lambda i: (i, 0))],
        out_specs=[
            pl.BlockSpec((blk_r, 1), lambda i: (i, 0)),
            pl.BlockSpec((blk_r, 1), lambda i: (i, 0)),
        ],
        out_shape=[
            jax.ShapeDtypeStruct((n, 1), affinity.dtype),
            jax.ShapeDtypeStruct((n, 1), jnp.int32),
        ],
    )(affinity)

    out, _ = pl.pallas_call(
        _col_mask_kernel,
        grid=(n // blk_c,),
        in_specs=[
            pl.BlockSpec((n, blk_c), lambda j: (0, j)),
            pl.BlockSpec((n, 1), lambda j: (0, 0)),
            pl.BlockSpec((n, 1), lambda j: (0, 0)),
        ],
        out_specs=[
            pl.BlockSpec((n, blk_c), lambda j: (0, j)),
            pl.BlockSpec((1, blk_c), lambda j: (0, j)),
        ],
        out_shape=[
            jax.ShapeDtypeStruct((n, n), affinity.dtype),
            jax.ShapeDtypeStruct((1, n), jnp.int32),
        ],
    )(affinity, rt, rji)

    return out
